# baseline (speedup 1.0000x reference)
import functools

import jax
import jax.numpy as jnp
from jax import lax
from jax.experimental import pallas as pl
from jax.experimental.pallas import tpu as pltpu
from jax.experimental.pallas import tpu_sc as plsc

_WSIZE = 1280000
_NROW, _NCOL = 100000, 128
_N = _NROW * _NCOL
_NC, _NS = 2, 16
_NW = _NC * _NS
_PER_W = _N // _NW
_NBUF = 5
_CHUNK = 3200
_NCHUNK = _PER_W // _CHUNK
_QUADS = _NCHUNK // _NBUF


def _roast_body(w_hbm, idx_hbm, g_hbm, out_hbm, *scratch):
    idx = scratch[0:_NBUF]
    g = scratch[_NBUF:2 * _NBUF]
    val = scratch[2 * _NBUF:3 * _NBUF]
    table = scratch[3 * _NBUF]
    sin = scratch[3 * _NBUF + 1:3 * _NBUF + 1 + _NBUF]
    sg = scratch[3 * _NBUF + 1 + _NBUF:3 * _NBUF + 1 + 2 * _NBUF]
    so = scratch[3 * _NBUF + 1 + 2 * _NBUF:3 * _NBUF + 1 + 3 * _NBUF]

    cid = lax.axis_index("c")
    sid = lax.axis_index("s")
    wid = sid * _NC + cid
    w0 = wid * _PER_W

    def issue_in(k, s):
        base = w0 + k * _CHUNK
        pltpu.async_copy(idx_hbm.at[pl.ds(base, _CHUNK)], idx[s], sin[s])
        pltpu.async_copy(g_hbm.at[pl.ds(base, _CHUNK)], g[s], sin[s])

    def wait_in(k, s):
        base = w0 + k * _CHUNK
        pltpu.make_async_copy(idx_hbm.at[pl.ds(base, _CHUNK)], idx[s], sin[s]).wait()
        pltpu.make_async_copy(g_hbm.at[pl.ds(base, _CHUNK)], g[s], sin[s]).wait()

    def issue_out(k, s):
        base = w0 + k * _CHUNK
        pltpu.async_copy(val[s], out_hbm.at[pl.ds(base, _CHUNK)], so[s])

    def wait_out(k, s):
        base = w0 + k * _CHUNK
        pltpu.make_async_copy(val[s], out_hbm.at[pl.ds(base, _CHUNK)], so[s]).wait()

    def issue_gather(s):
        pltpu.async_copy(table.at[idx[s]], val[s], sg[s])

    def wait_gather(s):
        pltpu.make_async_copy(table.at[idx[s]], val[s], sg[s]).wait()

    def multiply(s):
        val_v, g_v = val[s], g[s]

        @plsc.parallel_loop(0, _CHUNK, 16, unroll=8)
        def _(i):
            sl = pl.ds(i, 16)
            val_v[sl] = val_v[sl] * g_v[sl]

    for s in range(_NBUF):
        issue_in(s, s)
    seg = _WSIZE // _NS
    pltpu.sync_copy(
        w_hbm.at[pl.ds(sid * seg, seg)], table.at[pl.ds(sid * seg, seg)]
    )
    plsc.subcore_barrier()
    wait_in(0, 0)
    issue_gather(0)
    wait_in(1, 1)
    issue_gather(1)

    def quad_body(i, carry):
        for j in range(_NBUF):
            s = j
            k = _NBUF * i + j

            wait_gather(s)

            s2 = (j + 2) % _NBUF
            if j < _NBUF - 2:
                wait_in(k + 2, s2)

                @pl.when(i > 0)
                def _():
                    wait_out(k + 2 - _NBUF, s2)

                issue_gather(s2)
            else:

                @pl.when(i < _QUADS - 1)
                def _():
                    wait_in(k + 2, s2)
                    wait_out(k + 2 - _NBUF, s2)
                    issue_gather(s2)

            multiply(s)
            issue_out(k, s)

            @pl.when(i < _QUADS - 1)
            def _():
                issue_in(k + _NBUF, s)

        return carry

    lax.fori_loop(0, _QUADS, quad_body, 0)

    for s in range(_NBUF):
        wait_out(_NCHUNK - _NBUF + s, s)


def kernel(weight, IDX, G):
    mesh = plsc.VectorSubcoreMesh(
        core_axis_name="c", subcore_axis_name="s", num_cores=_NC,
        num_subcores=_NS,
    )
    scratch = (
        [pltpu.VMEM((_CHUNK,), jnp.int32) for _ in range(_NBUF)]
        + [pltpu.VMEM((_CHUNK,), jnp.float32) for _ in range(_NBUF)]
        + [pltpu.VMEM((_CHUNK,), jnp.float32) for _ in range(_NBUF)]
        + [pltpu.VMEM_SHARED((_WSIZE,), jnp.float32)]
        + [pltpu.SemaphoreType.DMA for _ in range(3 * _NBUF)]
    )
    roast = pl.kernel(
        _roast_body,
        out_type=jax.ShapeDtypeStruct((_N,), jnp.float32),
        mesh=mesh,
        scratch_types=scratch,
    )
    out = roast(weight, IDX.reshape(-1), G.reshape(-1))
    return out.reshape(_NROW, _NCOL)
